# Initial kernel scaffold; baseline (speedup 1.0000x reference)
import jax
import jax.numpy as jnp
from jax.experimental import pallas as pl


def kernel(x, token_table, pos_table):
    raise NotImplementedError("write your pallas kernel here")


# speedup vs baseline: 1.3203x; 1.3203x over previous
import functools

import jax
import jax.numpy as jnp
from jax import lax
from jax.experimental import pallas as pl
from jax.experimental.pallas import tpu as pltpu
from jax.experimental.pallas import tpu_sc as plsc

NC = 2
NS = 16
NW = NC * NS
LANES = 16


@functools.lru_cache(maxsize=None)
def _build(B, L, V, D):
    FLAT = B * L
    per_w = FLAT // NW
    CB = 8
    C = CB * L
    n_chunks = per_w // C
    assert per_w % C == 0 and FLAT % NW == 0 and D == 2 * LANES

    mesh = plsc.VectorSubcoreMesh(core_axis_name="c", subcore_axis_name="s")

    @functools.partial(
        pl.kernel,
        mesh=mesh,
        compiler_params=pltpu.CompilerParams(use_tc_tiling_on_sc=False),
        out_type=jax.ShapeDtypeStruct((FLAT, D), jnp.float32),
        scratch_types=[
            pltpu.VMEM((C,), jnp.int32),
            pltpu.VMEM((C, D), jnp.float32),
            pltpu.VMEM((L, D), jnp.float32),
            pltpu.SemaphoreType.DMA,
        ],
    )
    def k(tok_hbm, idx_hbm, pos_hbm, out_hbm, idx_v, rows_v, pos_v, sem):
        wid = lax.axis_index("s") * NC + lax.axis_index("c")
        base = wid * per_w
        pltpu.sync_copy(pos_hbm, pos_v)

        def chunk_body(ci, _):
            off = base + ci * C
            pltpu.sync_copy(idx_hbm.at[pl.ds(off, C)], idx_v)
            pltpu.async_copy(tok_hbm.at[idx_v], rows_v, sem).wait()

            def add_b(bi, _):
                rbase = bi * L

                def add_l(li, _):
                    r = rbase + li
                    rows_v[r, pl.ds(0, LANES)] = (
                        rows_v[r, pl.ds(0, LANES)] + pos_v[li, pl.ds(0, LANES)])
                    rows_v[r, pl.ds(LANES, LANES)] = (
                        rows_v[r, pl.ds(LANES, LANES)] + pos_v[li, pl.ds(LANES, LANES)])
                    return 0

                return lax.fori_loop(0, L, add_l, 0)

            lax.fori_loop(0, CB, add_b, 0)
            pltpu.sync_copy(rows_v, out_hbm.at[pl.ds(off, C)])
            return 0

        lax.fori_loop(0, n_chunks, chunk_body, 0)

    return k


def kernel(x, token_table, pos_table):
    B, L = x.shape
    V, D = token_table.shape
    k = _build(B, L, V, D)
    out_flat = k(token_table, x.reshape(-1).astype(jnp.int32), pos_table)
    return out_flat.reshape(B, L, D)


# speedup vs baseline: 1.4921x; 1.1301x over previous
import functools

import jax
import jax.numpy as jnp
from jax import lax
from jax.experimental import pallas as pl
from jax.experimental.pallas import tpu as pltpu
from jax.experimental.pallas import tpu_sc as plsc

NC = 2
NS = 16
NW = NC * NS
LANES = 16
NBUF = 4


@functools.lru_cache(maxsize=None)
def _build(B, L, V, D):
    FLAT = B * L
    per_w = FLAT // NW
    CB = 4
    C = CB * L
    n_chunks = per_w // C
    assert per_w % C == 0 and FLAT % NW == 0 and D == 2 * LANES
    assert n_chunks % NBUF == 0 and n_chunks >= 2 * NBUF

    mesh = plsc.VectorSubcoreMesh(core_axis_name="c", subcore_axis_name="s")

    @functools.partial(
        pl.kernel,
        mesh=mesh,
        compiler_params=pltpu.CompilerParams(use_tc_tiling_on_sc=False),
        out_type=jax.ShapeDtypeStruct((FLAT, D), jnp.float32),
        scratch_types=(
            [pltpu.VMEM((C,), jnp.int32) for _ in range(NBUF)]
            + [pltpu.VMEM((C, D), jnp.float32) for _ in range(NBUF)]
            + [pltpu.VMEM((L, D), jnp.float32)]
            + [pltpu.SemaphoreType.DMA for _ in range(2 * NBUF)]
        ),
    )
    def k(tok_hbm, idx_hbm, pos_hbm, out_hbm, *refs):
        idx_v = refs[0:NBUF]
        rows_v = refs[NBUF:2 * NBUF]
        pos_v = refs[2 * NBUF]
        gsem = refs[2 * NBUF + 1:2 * NBUF + 1 + NBUF]
        ssem = refs[2 * NBUF + 1 + NBUF:2 * NBUF + 1 + 2 * NBUF]

        wid = lax.axis_index("s") * NC + lax.axis_index("c")
        base = wid * per_w
        pltpu.sync_copy(pos_hbm, pos_v)

        def gather(ci, b):
            off = base + ci * C
            pltpu.sync_copy(idx_hbm.at[pl.ds(off, C)], idx_v[b])
            pltpu.make_async_copy(tok_hbm.at[idx_v[b]], rows_v[b], gsem[b]).start()

        gather(0, 0)
        gather(1, 1)

        def quad_body(pi, _):
            for b in range(NBUF):
                ci = NBUF * pi + b
                off = base + ci * C
                pltpu.make_async_copy(
                    tok_hbm.at[idx_v[b]], rows_v[b], gsem[b]).wait()

                def add_l(li, _):
                    p0 = pos_v[li, pl.ds(0, LANES)]
                    p1 = pos_v[li, pl.ds(LANES, LANES)]
                    for cb in range(CB):
                        r = cb * L + li
                        rows_v[b][r, pl.ds(0, LANES)] = (
                            rows_v[b][r, pl.ds(0, LANES)] + p0)
                        rows_v[b][r, pl.ds(LANES, LANES)] = (
                            rows_v[b][r, pl.ds(LANES, LANES)] + p1)
                    return 0

                lax.fori_loop(0, L, add_l, 0, unroll=4)
                pltpu.make_async_copy(
                    rows_v[b], out_hbm.at[pl.ds(off, C)], ssem[b]).start()

                b2 = (b + 2) % NBUF

                @pl.when(ci >= 2)
                def _():
                    off_prev = base + (ci - 2) * C
                    pltpu.make_async_copy(
                        rows_v[b2], out_hbm.at[pl.ds(off_prev, C)],
                        ssem[b2]).wait()

                @pl.when(ci + 2 < n_chunks)
                def _():
                    off_next = base + (ci + 2) * C
                    pltpu.sync_copy(idx_hbm.at[pl.ds(off_next, C)], idx_v[b2])
                    pltpu.make_async_copy(
                        tok_hbm.at[idx_v[b2]], rows_v[b2], gsem[b2]).start()

            return 0

        lax.fori_loop(0, n_chunks // NBUF, quad_body, 0)

        for ci in (n_chunks - 2, n_chunks - 1):
            b = ci % NBUF
            off = base + ci * C
            pltpu.make_async_copy(
                rows_v[b], out_hbm.at[pl.ds(off, C)], ssem[b]).wait()

    return k


def kernel(x, token_table, pos_table):
    B, L = x.shape
    V, D = token_table.shape
    k = _build(B, L, V, D)
    out_flat = k(token_table, x.reshape(-1).astype(jnp.int32), pos_table)
    return out_flat.reshape(B, L, D)
